# baseline (speedup 1.0000x reference)
import jax
import jax.numpy as jnp
from jax import lax
from jax.experimental import pallas as pl
from jax.experimental.pallas import tpu as pltpu
from jax.experimental.pallas import tpu_sc as plsc

N = 10000
E = 320000
D = 128
OUT = 128

NC = 2
NS = 16
NW = NC * NS

EDGES_PER_W = E // NW
CHUNK = 80
NCHUNKS = EDGES_PER_W // CHUNK
NP = 10240
ROWS_PER_TILE = NP // NS
HR = NP // 128


NT = NCHUNKS // 3
NREM = NCHUNKS - 3 * NT


def _sc_body(x_hbm, src_hbm, dst_hbm, zrow_hbm,
             agg_hbm, hist_hbm,
             srcs, dsts, rows0, rows1, rows2, hist_v,
             acc_sh,
             semi, semg0, semg1, semg2, sems0, sems1, sems2, semz):
    c = lax.axis_index("c")
    s = lax.axis_index("s")
    w = c * NS + s
    base = w * EDGES_PER_W
    row0 = s * ROWS_PER_TILE

    def load_triple_idx(t, sl):
        for q in range(3):
            off = base + (3 * t + q) * CHUNK
            pltpu.async_copy(src_hbm.at[pl.ds(off, CHUNK)], srcs.at[3 * sl + q], semi)
            pltpu.async_copy(dst_hbm.at[pl.ds(off, CHUNK)], dsts.at[3 * sl + q], semi)

    def drain_triple_idx(sl):
        for q in range(3):
            pltpu.make_async_copy(src_hbm.at[pl.ds(base, CHUNK)],
                                  srcs.at[3 * sl + q], semi).wait()
            pltpu.make_async_copy(dst_hbm.at[pl.ds(base, CHUNK)],
                                  dsts.at[3 * sl + q], semi).wait()

    load_triple_idx(0, 0)
    pltpu.sync_copy(zrow_hbm, rows2)
    zd = []
    for k in range(ROWS_PER_TILE // CHUNK):
        zd.append(pltpu.async_copy(
            rows2, acc_sh.at[pl.ds(row0 + k * CHUNK, CHUNK), :], semz))

    zeros16 = jnp.zeros((16,), jnp.float32)

    def zstep(i, carry):
        for g in range(128 // 16):
            hist_v[i, pl.ds(g * 16, 16)] = zeros16
        return carry

    lax.fori_loop(0, HR, zstep, 0)
    for d in zd:
        d.wait()
    plsc.subcore_barrier()

    ones16 = jnp.ones((16,), jnp.float32)

    def hist_update(dref):
        for g in range(CHUNK // 16):
            ig = dref[pl.ds(g * 16, 16)]
            plsc.addupdate_scatter(
                hist_v, [lax.shift_right_logical(ig, 7),
                         lax.bitwise_and(ig, 127)], ones16)

    def step(i, carry):
        sl = lax.bitwise_and(i, 1)
        s0 = 3 * sl
        drain_triple_idx(sl)
        nxt = lax.rem(i + 1, NT)
        load_triple_idx(nxt, 1 - sl)
        g0 = pltpu.async_copy(x_hbm.at[srcs.at[s0]], rows0, semg0)
        g1 = pltpu.async_copy(x_hbm.at[srcs.at[s0 + 1]], rows1, semg1)
        g2 = pltpu.async_copy(x_hbm.at[srcs.at[s0 + 2]], rows2, semg2)
        g0.wait()
        sc0 = pltpu.async_copy(rows0, acc_sh.at[dsts.at[s0]], sems0, add=True)
        hist_update(dsts.at[s0])
        g1.wait()
        sc1 = pltpu.async_copy(rows1, acc_sh.at[dsts.at[s0 + 1]], sems1, add=True)
        hist_update(dsts.at[s0 + 1])
        g2.wait()
        sc2 = pltpu.async_copy(rows2, acc_sh.at[dsts.at[s0 + 2]], sems2, add=True)
        hist_update(dsts.at[s0 + 2])
        sc0.wait()
        sc1.wait()
        sc2.wait()
        return carry

    lax.fori_loop(0, NT, step, 0)
    drain_triple_idx(lax.bitwise_and(NT, 1))

    offA = base + 3 * NT * CHUNK
    offB = offA + CHUNK
    pltpu.sync_copy(src_hbm.at[pl.ds(offA, CHUNK)], srcs.at[0])
    pltpu.sync_copy(dst_hbm.at[pl.ds(offA, CHUNK)], dsts.at[0])
    pltpu.sync_copy(src_hbm.at[pl.ds(offB, CHUNK)], srcs.at[1])
    pltpu.sync_copy(dst_hbm.at[pl.ds(offB, CHUNK)], dsts.at[1])
    ga = pltpu.async_copy(x_hbm.at[srcs.at[0]], rows0, semg0)
    gb = pltpu.async_copy(x_hbm.at[srcs.at[1]], rows1, semg1)
    ga.wait()
    sa = pltpu.async_copy(rows0, acc_sh.at[dsts.at[0]], sems0, add=True)
    hist_update(dsts.at[0])
    gb.wait()
    sb = pltpu.async_copy(rows1, acc_sh.at[dsts.at[1]], sems1, add=True)
    hist_update(dsts.at[1])
    sa.wait()
    sb.wait()

    plsc.subcore_barrier()

    hd = pltpu.async_copy(hist_v, hist_hbm.at[w], semz)
    nk = ROWS_PER_TILE // CHUNK
    bufs = [rows0, rows1]
    isems = [semg0, semg1]
    osems = [sems0, sems1]
    din = {0: pltpu.async_copy(acc_sh.at[pl.ds(row0, CHUNK), :], bufs[0],
                               isems[0])}
    dout = {}
    for k in range(nk):
        b = k % 2
        din[k].wait()
        if k + 1 < nk:
            if k + 1 >= 2:
                dout[k - 1].wait()
            din[k + 1] = pltpu.async_copy(
                acc_sh.at[pl.ds(row0 + (k + 1) * CHUNK, CHUNK), :],
                bufs[(k + 1) % 2], isems[(k + 1) % 2])
        dout[k] = pltpu.async_copy(
            bufs[b], agg_hbm.at[c, pl.ds(row0 + k * CHUNK, CHUNK), :],
            osems[b])
    dout[nk - 2].wait()
    dout[nk - 1].wait()
    hd.wait()


@jax.jit
def _sc_segment_sum(x, src, dst):
    zrow = jnp.zeros((CHUNK, D), jnp.float32)
    mesh = plsc.VectorSubcoreMesh(core_axis_name="c", subcore_axis_name="s",
                                  num_cores=NC, num_subcores=NS)
    fn = pl.kernel(
        _sc_body,
        out_type=(jax.ShapeDtypeStruct((NC, NP, D), jnp.float32),
                  jax.ShapeDtypeStruct((NW, HR, 128), jnp.float32)),
        mesh=mesh,
        compiler_params=pltpu.CompilerParams(needs_layout_passes=False),
        scratch_types=[
            pltpu.VMEM((6, CHUNK), jnp.int32),
            pltpu.VMEM((6, CHUNK), jnp.int32),
            pltpu.VMEM((CHUNK, D), jnp.float32),
            pltpu.VMEM((CHUNK, D), jnp.float32),
            pltpu.VMEM((CHUNK, D), jnp.float32),
            pltpu.VMEM((HR, 128), jnp.float32),
            pltpu.VMEM_SHARED((NP, D), jnp.float32),
            pltpu.SemaphoreType.DMA,
            pltpu.SemaphoreType.DMA,
            pltpu.SemaphoreType.DMA,
            pltpu.SemaphoreType.DMA,
            pltpu.SemaphoreType.DMA,
            pltpu.SemaphoreType.DMA,
            pltpu.SemaphoreType.DMA,
            pltpu.SemaphoreType.DMA,
        ],
    )
    return fn(x, src, dst, zrow)


def _tc_body(x_ref, a_ref, h_ref, wl_ref, wr_ref, b_ref, h_out, c_out):
    cnt_row = jnp.sum(h_ref[...], axis=0, keepdims=True)
    inv_row = 1.0 / jnp.maximum(cnt_row, 1.0)
    inv_col = lax.dot_general(inv_row, jnp.ones((1, D), jnp.float32),
                              ((( 0,), (0,)), ((), ())),
                              precision=lax.Precision.HIGHEST)
    agg = (a_ref[0] + a_ref[1]) * inv_col
    z = (jnp.dot(agg, wl_ref[...], preferred_element_type=jnp.float32)
         + jnp.dot(x_ref[...], wr_ref[...], preferred_element_type=jnp.float32)
         + b_ref[...])
    i_g = jax.nn.sigmoid(z[:, :OUT])
    t_g = jnp.tanh(z[:, OUT:2 * OUT])
    o_g = jax.nn.sigmoid(z[:, 2 * OUT:])
    c_new = i_g * t_g
    h_out[...] = o_g * jnp.tanh(c_new)
    c_out[...] = c_new


BLK = 1024
HRB = BLK // 128


@jax.jit
def _tc_gates(x, agg2, hist, wl3, wr3, b3):
    grid = (NP // BLK,)
    return pl.pallas_call(
        _tc_body,
        grid=grid,
        in_specs=[
            pl.BlockSpec((BLK, D), lambda i: (i, 0)),
            pl.BlockSpec((NC, BLK, D), lambda i: (0, i, 0)),
            pl.BlockSpec((NW, BLK), lambda i: (0, i)),
            pl.BlockSpec((D, 3 * OUT), lambda i: (0, 0)),
            pl.BlockSpec((D, 3 * OUT), lambda i: (0, 0)),
            pl.BlockSpec((1, 3 * OUT), lambda i: (0, 0)),
        ],
        out_specs=[
            pl.BlockSpec((BLK, OUT), lambda i: (i, 0)),
            pl.BlockSpec((BLK, OUT), lambda i: (i, 0)),
        ],
        out_shape=[
            jax.ShapeDtypeStruct((N, OUT), jnp.float32),
            jax.ShapeDtypeStruct((N, OUT), jnp.float32),
        ],
    )(x, agg2, hist, wl3, wr3, b3)


def kernel(x, edge_index, Wl_i, Wr_i, b_i, Wl_f, Wr_f, b_f,
           Wl_c, Wr_c, b_c, Wl_o, Wr_o, b_o):
    agg2, hist = _sc_segment_sum(x, edge_index[0], edge_index[1])
    hist = hist.reshape(NW, NP)
    wl3 = jnp.concatenate([Wl_i, Wl_c, Wl_o], axis=1)
    wr3 = jnp.concatenate([Wr_i, Wr_c, Wr_o], axis=1)
    b3 = jnp.concatenate([b_i, b_c, b_o]).reshape(1, 3 * OUT)
    h_new, c_new = _tc_gates(x, agg2, hist, wl3, wr3, b3)
    return (h_new, c_new)


# speedup vs baseline: 1.0149x; 1.0149x over previous
import jax
import jax.numpy as jnp
from jax import lax
from jax.experimental import pallas as pl
from jax.experimental.pallas import tpu as pltpu
from jax.experimental.pallas import tpu_sc as plsc

N = 10000
E = 320000
D = 128
OUT = 128

NC = 2
NS = 16
NW = NC * NS

EDGES_PER_W = E // NW
CHUNK = 80
NCHUNKS = EDGES_PER_W // CHUNK
NP = 10240
ROWS_PER_TILE = NP // NS
HR = NP // 128


NT = NCHUNKS // 3
NREM = NCHUNKS - 3 * NT


def _sc_body(x_hbm, src_hbm, dst_hbm, zrow_hbm,
             agg_hbm, hist_hbm,
             srcs, dsts, rows0, rows1, rows2, hist_v,
             acc_sh,
             semi, semg0, semg1, semg2, sems0, sems1, sems2, semz):
    c = lax.axis_index("c")
    s = lax.axis_index("s")
    w = c * NS + s
    base = w * EDGES_PER_W
    row0 = s * ROWS_PER_TILE

    def load_triple_idx(t, sl):
        for q in range(3):
            off = base + (3 * t + q) * CHUNK
            pltpu.async_copy(src_hbm.at[pl.ds(off, CHUNK)], srcs.at[3 * sl + q], semi)
            pltpu.async_copy(dst_hbm.at[pl.ds(off, CHUNK)], dsts.at[3 * sl + q], semi)

    def drain_triple_idx(sl):
        for q in range(3):
            pltpu.make_async_copy(src_hbm.at[pl.ds(base, CHUNK)],
                                  srcs.at[3 * sl + q], semi).wait()
            pltpu.make_async_copy(dst_hbm.at[pl.ds(base, CHUNK)],
                                  dsts.at[3 * sl + q], semi).wait()

    load_triple_idx(0, 0)
    pltpu.sync_copy(zrow_hbm, rows2)
    zd = []
    for k in range(ROWS_PER_TILE // CHUNK):
        zd.append(pltpu.async_copy(
            rows2, acc_sh.at[pl.ds(row0 + k * CHUNK, CHUNK), :], semz))

    zeros16 = jnp.zeros((16,), jnp.float32)

    def zstep(i, carry):
        for g in range(128 // 16):
            hist_v[i, pl.ds(g * 16, 16)] = zeros16
        return carry

    lax.fori_loop(0, HR, zstep, 0)
    for d in zd:
        d.wait()
    plsc.subcore_barrier()

    ones16 = jnp.ones((16,), jnp.float32)

    def hist_update(dref):
        for g in range(CHUNK // 16):
            ig = dref[pl.ds(g * 16, 16)]
            plsc.addupdate_scatter(
                hist_v, [lax.shift_right_logical(ig, 7),
                         lax.bitwise_and(ig, 127)], ones16)

    def step(i, carry):
        sl = lax.bitwise_and(i, 1)
        s0 = 3 * sl
        drain_triple_idx(sl)
        nxt = lax.rem(i + 1, NT)
        load_triple_idx(nxt, 1 - sl)
        g0 = pltpu.async_copy(x_hbm.at[srcs.at[s0]], rows0, semg0)
        g1 = pltpu.async_copy(x_hbm.at[srcs.at[s0 + 1]], rows1, semg1)
        g2 = pltpu.async_copy(x_hbm.at[srcs.at[s0 + 2]], rows2, semg2)
        g0.wait()
        sc0 = pltpu.async_copy(rows0, acc_sh.at[dsts.at[s0]], sems0, add=True)
        hist_update(dsts.at[s0])
        g1.wait()
        sc1 = pltpu.async_copy(rows1, acc_sh.at[dsts.at[s0 + 1]], sems1, add=True)
        hist_update(dsts.at[s0 + 1])
        g2.wait()
        sc2 = pltpu.async_copy(rows2, acc_sh.at[dsts.at[s0 + 2]], sems2, add=True)
        hist_update(dsts.at[s0 + 2])
        sc0.wait()
        sc1.wait()
        sc2.wait()
        return carry

    lax.fori_loop(0, NT, step, 0)
    drain_triple_idx(lax.bitwise_and(NT, 1))

    offA = base + 3 * NT * CHUNK
    offB = offA + CHUNK
    pltpu.sync_copy(src_hbm.at[pl.ds(offA, CHUNK)], srcs.at[0])
    pltpu.sync_copy(dst_hbm.at[pl.ds(offA, CHUNK)], dsts.at[0])
    pltpu.sync_copy(src_hbm.at[pl.ds(offB, CHUNK)], srcs.at[1])
    pltpu.sync_copy(dst_hbm.at[pl.ds(offB, CHUNK)], dsts.at[1])
    ga = pltpu.async_copy(x_hbm.at[srcs.at[0]], rows0, semg0)
    gb = pltpu.async_copy(x_hbm.at[srcs.at[1]], rows1, semg1)
    ga.wait()
    sa = pltpu.async_copy(rows0, acc_sh.at[dsts.at[0]], sems0, add=True)
    hist_update(dsts.at[0])
    gb.wait()
    sb = pltpu.async_copy(rows1, acc_sh.at[dsts.at[1]], sems1, add=True)
    hist_update(dsts.at[1])
    sa.wait()
    sb.wait()

    plsc.subcore_barrier()

    hd = pltpu.async_copy(hist_v, hist_hbm.at[w], semz)
    nk = ROWS_PER_TILE // CHUNK
    bufs = [rows0, rows1]
    isems = [semg0, semg1]
    osems = [sems0, sems1]
    din = {0: pltpu.async_copy(acc_sh.at[pl.ds(row0, CHUNK), :], bufs[0],
                               isems[0])}
    dout = {}
    for k in range(nk):
        b = k % 2
        din[k].wait()
        if k + 1 < nk:
            if k + 1 >= 2:
                dout[k - 1].wait()
            din[k + 1] = pltpu.async_copy(
                acc_sh.at[pl.ds(row0 + (k + 1) * CHUNK, CHUNK), :],
                bufs[(k + 1) % 2], isems[(k + 1) % 2])
        dout[k] = pltpu.async_copy(
            bufs[b], agg_hbm.at[c, pl.ds(row0 + k * CHUNK, CHUNK), :],
            osems[b])
    dout[nk - 2].wait()
    dout[nk - 1].wait()
    hd.wait()


@jax.jit
def _sc_segment_sum(x, src, dst):
    zrow = jnp.zeros((CHUNK, D), jnp.float32)
    mesh = plsc.VectorSubcoreMesh(core_axis_name="c", subcore_axis_name="s",
                                  num_cores=NC, num_subcores=NS)
    fn = pl.kernel(
        _sc_body,
        out_type=(jax.ShapeDtypeStruct((NC, NP, D), jnp.float32),
                  jax.ShapeDtypeStruct((NW, HR, 128), jnp.float32)),
        mesh=mesh,
        compiler_params=pltpu.CompilerParams(needs_layout_passes=False),
        scratch_types=[
            pltpu.VMEM((6, CHUNK), jnp.int32),
            pltpu.VMEM((6, CHUNK), jnp.int32),
            pltpu.VMEM((CHUNK, D), jnp.float32),
            pltpu.VMEM((CHUNK, D), jnp.float32),
            pltpu.VMEM((CHUNK, D), jnp.float32),
            pltpu.VMEM((HR, 128), jnp.float32),
            pltpu.VMEM_SHARED((NP, D), jnp.float32),
            pltpu.SemaphoreType.DMA,
            pltpu.SemaphoreType.DMA,
            pltpu.SemaphoreType.DMA,
            pltpu.SemaphoreType.DMA,
            pltpu.SemaphoreType.DMA,
            pltpu.SemaphoreType.DMA,
            pltpu.SemaphoreType.DMA,
            pltpu.SemaphoreType.DMA,
        ],
    )
    return fn(x, src, dst, zrow)


def _tc_body(x_ref, a_ref, h_ref, wl_ref, wr_ref, b_ref, h_out, c_out):
    cnt_row = jnp.sum(h_ref[...], axis=0, keepdims=True)
    inv_row = 1.0 / jnp.maximum(cnt_row, 1.0)
    inv_col = lax.dot_general(inv_row, jnp.ones((1, D), jnp.float32),
                              ((( 0,), (0,)), ((), ())),
                              precision=lax.Precision.HIGHEST)
    agg = (a_ref[0] + a_ref[1]) * inv_col
    z = (jnp.dot(agg, wl_ref[...], preferred_element_type=jnp.float32)
         + jnp.dot(x_ref[...], wr_ref[...], preferred_element_type=jnp.float32)
         + b_ref[...])
    i_g = jax.nn.sigmoid(z[:, :OUT])
    t_g = jnp.tanh(z[:, OUT:2 * OUT])
    o_g = jax.nn.sigmoid(z[:, 2 * OUT:])
    c_new = i_g * t_g
    h_out[...] = o_g * jnp.tanh(c_new)
    c_out[...] = c_new


BLK = 2048
HRB = BLK // 128


@jax.jit
def _tc_gates(x, agg2, hist, wl3, wr3, b3):
    grid = (NP // BLK,)
    return pl.pallas_call(
        _tc_body,
        grid=grid,
        in_specs=[
            pl.BlockSpec((BLK, D), lambda i: (i, 0)),
            pl.BlockSpec((NC, BLK, D), lambda i: (0, i, 0)),
            pl.BlockSpec((NW, BLK), lambda i: (0, i)),
            pl.BlockSpec((D, 3 * OUT), lambda i: (0, 0)),
            pl.BlockSpec((D, 3 * OUT), lambda i: (0, 0)),
            pl.BlockSpec((1, 3 * OUT), lambda i: (0, 0)),
        ],
        out_specs=[
            pl.BlockSpec((BLK, OUT), lambda i: (i, 0)),
            pl.BlockSpec((BLK, OUT), lambda i: (i, 0)),
        ],
        out_shape=[
            jax.ShapeDtypeStruct((N, OUT), jnp.float32),
            jax.ShapeDtypeStruct((N, OUT), jnp.float32),
        ],
    )(x, agg2, hist, wl3, wr3, b3)


def kernel(x, edge_index, Wl_i, Wr_i, b_i, Wl_f, Wr_f, b_f,
           Wl_c, Wr_c, b_c, Wl_o, Wr_o, b_o):
    agg2, hist = _sc_segment_sum(x, edge_index[0], edge_index[1])
    hist = hist.reshape(NW, NP)
    wl3 = jnp.concatenate([Wl_i, Wl_c, Wl_o], axis=1)
    wr3 = jnp.concatenate([Wr_i, Wr_c, Wr_o], axis=1)
    b3 = jnp.concatenate([b_i, b_c, b_o]).reshape(1, 3 * OUT)
    h_new, c_new = _tc_gates(x, agg2, hist, wl3, wr3, b3)
    return (h_new, c_new)


# speedup vs baseline: 1.0158x; 1.0009x over previous
import jax
import jax.numpy as jnp
from jax import lax
from jax.experimental import pallas as pl
from jax.experimental.pallas import tpu as pltpu
from jax.experimental.pallas import tpu_sc as plsc

N = 10000
E = 320000
D = 128
OUT = 128

NC = 2
NS = 16
NW = NC * NS

EDGES_PER_W = E // NW
CHUNK = 80
NCHUNKS = EDGES_PER_W // CHUNK
NP = 10240
ROWS_PER_TILE = NP // NS
HR = NP // 128


NT = NCHUNKS // 3
NREM = NCHUNKS - 3 * NT


def _sc_body(x_hbm, src_hbm, dst_hbm, zrow_hbm,
             agg_hbm, hist_hbm,
             srcs, dsts, rows0, rows1, rows2, hist_v,
             acc_sh,
             semi, semg0, semg1, semg2, sems0, sems1, sems2, semz):
    c = lax.axis_index("c")
    s = lax.axis_index("s")
    w = c * NS + s
    base = w * EDGES_PER_W
    row0 = s * ROWS_PER_TILE

    def load_triple_idx(t, sl):
        for q in range(3):
            off = base + (3 * t + q) * CHUNK
            pltpu.async_copy(src_hbm.at[pl.ds(off, CHUNK)], srcs.at[3 * sl + q], semi)
            pltpu.async_copy(dst_hbm.at[pl.ds(off, CHUNK)], dsts.at[3 * sl + q], semi)

    def drain_triple_idx(sl):
        for q in range(3):
            pltpu.make_async_copy(src_hbm.at[pl.ds(base, CHUNK)],
                                  srcs.at[3 * sl + q], semi).wait()
            pltpu.make_async_copy(dst_hbm.at[pl.ds(base, CHUNK)],
                                  dsts.at[3 * sl + q], semi).wait()

    load_triple_idx(0, 0)
    pltpu.sync_copy(zrow_hbm, rows2)
    zd = []
    for k in range(ROWS_PER_TILE // CHUNK):
        zd.append(pltpu.async_copy(
            rows2, acc_sh.at[pl.ds(row0 + k * CHUNK, CHUNK), :], semz))

    zeros16 = jnp.zeros((16,), jnp.float32)

    def zstep(i, carry):
        for g in range(128 // 16):
            hist_v[i, pl.ds(g * 16, 16)] = zeros16
        return carry

    lax.fori_loop(0, HR, zstep, 0)
    for d in zd:
        d.wait()
    plsc.subcore_barrier()

    ones16 = jnp.ones((16,), jnp.float32)

    def hist_update(dref):
        for g in range(CHUNK // 16):
            ig = dref[pl.ds(g * 16, 16)]
            plsc.addupdate_scatter(
                hist_v, [lax.shift_right_logical(ig, 7),
                         lax.bitwise_and(ig, 127)], ones16)

    def step(i, carry):
        sl = lax.bitwise_and(i, 1)
        s0 = 3 * sl
        drain_triple_idx(sl)
        nxt = lax.rem(i + 1, NT)
        load_triple_idx(nxt, 1 - sl)
        g0 = pltpu.async_copy(x_hbm.at[srcs.at[s0]], rows0, semg0)
        g1 = pltpu.async_copy(x_hbm.at[srcs.at[s0 + 1]], rows1, semg1)
        g2 = pltpu.async_copy(x_hbm.at[srcs.at[s0 + 2]], rows2, semg2)
        g0.wait()
        sc0 = pltpu.async_copy(rows0, acc_sh.at[dsts.at[s0]], sems0, add=True)
        hist_update(dsts.at[s0])
        g1.wait()
        sc1 = pltpu.async_copy(rows1, acc_sh.at[dsts.at[s0 + 1]], sems1, add=True)
        hist_update(dsts.at[s0 + 1])
        g2.wait()
        sc2 = pltpu.async_copy(rows2, acc_sh.at[dsts.at[s0 + 2]], sems2, add=True)
        hist_update(dsts.at[s0 + 2])
        sc0.wait()
        sc1.wait()
        sc2.wait()
        return carry

    lax.fori_loop(0, NT, step, 0)
    drain_triple_idx(lax.bitwise_and(NT, 1))

    offA = base + 3 * NT * CHUNK
    offB = offA + CHUNK
    pltpu.sync_copy(src_hbm.at[pl.ds(offA, CHUNK)], srcs.at[0])
    pltpu.sync_copy(dst_hbm.at[pl.ds(offA, CHUNK)], dsts.at[0])
    pltpu.sync_copy(src_hbm.at[pl.ds(offB, CHUNK)], srcs.at[1])
    pltpu.sync_copy(dst_hbm.at[pl.ds(offB, CHUNK)], dsts.at[1])
    ga = pltpu.async_copy(x_hbm.at[srcs.at[0]], rows0, semg0)
    gb = pltpu.async_copy(x_hbm.at[srcs.at[1]], rows1, semg1)
    ga.wait()
    sa = pltpu.async_copy(rows0, acc_sh.at[dsts.at[0]], sems0, add=True)
    hist_update(dsts.at[0])
    gb.wait()
    sb = pltpu.async_copy(rows1, acc_sh.at[dsts.at[1]], sems1, add=True)
    hist_update(dsts.at[1])
    sa.wait()
    sb.wait()

    plsc.subcore_barrier()

    hd = pltpu.async_copy(hist_v, hist_hbm.at[w], semz)
    nk = ROWS_PER_TILE // CHUNK
    bufs = [rows0, rows1]
    isems = [semg0, semg1]
    osems = [sems0, sems1]
    din = {0: pltpu.async_copy(acc_sh.at[pl.ds(row0, CHUNK), :], bufs[0],
                               isems[0])}
    dout = {}
    for k in range(nk):
        b = k % 2
        din[k].wait()
        if k + 1 < nk:
            if k + 1 >= 2:
                dout[k - 1].wait()
            din[k + 1] = pltpu.async_copy(
                acc_sh.at[pl.ds(row0 + (k + 1) * CHUNK, CHUNK), :],
                bufs[(k + 1) % 2], isems[(k + 1) % 2])
        dout[k] = pltpu.async_copy(
            bufs[b], agg_hbm.at[c, pl.ds(row0 + k * CHUNK, CHUNK), :],
            osems[b])
    dout[nk - 2].wait()
    dout[nk - 1].wait()
    hd.wait()


@jax.jit
def _sc_segment_sum(x, src, dst):
    zrow = jnp.zeros((CHUNK, D), jnp.float32)
    mesh = plsc.VectorSubcoreMesh(core_axis_name="c", subcore_axis_name="s",
                                  num_cores=NC, num_subcores=NS)
    fn = pl.kernel(
        _sc_body,
        out_type=(jax.ShapeDtypeStruct((NC, NP, D), jnp.float32),
                  jax.ShapeDtypeStruct((NW, HR, 128), jnp.float32)),
        mesh=mesh,
        compiler_params=pltpu.CompilerParams(needs_layout_passes=False),
        scratch_types=[
            pltpu.VMEM((6, CHUNK), jnp.int32),
            pltpu.VMEM((6, CHUNK), jnp.int32),
            pltpu.VMEM((CHUNK, D), jnp.float32),
            pltpu.VMEM((CHUNK, D), jnp.float32),
            pltpu.VMEM((CHUNK, D), jnp.float32),
            pltpu.VMEM((HR, 128), jnp.float32),
            pltpu.VMEM_SHARED((NP, D), jnp.float32),
            pltpu.SemaphoreType.DMA,
            pltpu.SemaphoreType.DMA,
            pltpu.SemaphoreType.DMA,
            pltpu.SemaphoreType.DMA,
            pltpu.SemaphoreType.DMA,
            pltpu.SemaphoreType.DMA,
            pltpu.SemaphoreType.DMA,
            pltpu.SemaphoreType.DMA,
        ],
    )
    return fn(x, src, dst, zrow)


def _tc_body(x_ref, a_ref, h_ref, wl_ref, wr_ref, b_ref, h_out, c_out):
    cnt_row = jnp.sum(h_ref[...], axis=0, keepdims=True)
    inv_row = 1.0 / jnp.maximum(cnt_row, 1.0)
    inv_col = lax.dot_general(inv_row, jnp.ones((1, D), jnp.float32),
                              ((( 0,), (0,)), ((), ())),
                              precision=lax.Precision.HIGHEST)
    agg = (a_ref[0] + a_ref[1]) * inv_col
    z = (jnp.dot(agg, wl_ref[...], preferred_element_type=jnp.float32)
         + jnp.dot(x_ref[...], wr_ref[...], preferred_element_type=jnp.float32)
         + b_ref[...])
    i_g = jax.nn.sigmoid(z[:, :OUT])
    t_g = jnp.tanh(z[:, OUT:2 * OUT])
    o_g = jax.nn.sigmoid(z[:, 2 * OUT:])
    c_new = i_g * t_g
    h_out[...] = o_g * jnp.tanh(c_new)
    c_out[...] = c_new


BLK = 2560
HRB = BLK // 128


@jax.jit
def _tc_gates(x, agg2, hist, wl3, wr3, b3):
    grid = (NP // BLK,)
    return pl.pallas_call(
        _tc_body,
        grid=grid,
        in_specs=[
            pl.BlockSpec((BLK, D), lambda i: (i, 0)),
            pl.BlockSpec((NC, BLK, D), lambda i: (0, i, 0)),
            pl.BlockSpec((NW, BLK), lambda i: (0, i)),
            pl.BlockSpec((D, 3 * OUT), lambda i: (0, 0)),
            pl.BlockSpec((D, 3 * OUT), lambda i: (0, 0)),
            pl.BlockSpec((1, 3 * OUT), lambda i: (0, 0)),
        ],
        out_specs=[
            pl.BlockSpec((BLK, OUT), lambda i: (i, 0)),
            pl.BlockSpec((BLK, OUT), lambda i: (i, 0)),
        ],
        out_shape=[
            jax.ShapeDtypeStruct((N, OUT), jnp.float32),
            jax.ShapeDtypeStruct((N, OUT), jnp.float32),
        ],
    )(x, agg2, hist, wl3, wr3, b3)


def kernel(x, edge_index, Wl_i, Wr_i, b_i, Wl_f, Wr_f, b_f,
           Wl_c, Wr_c, b_c, Wl_o, Wr_o, b_o):
    agg2, hist = _sc_segment_sum(x, edge_index[0], edge_index[1])
    hist = hist.reshape(NW, NP)
    wl3 = jnp.concatenate([Wl_i, Wl_c, Wl_o], axis=1)
    wr3 = jnp.concatenate([Wr_i, Wr_c, Wr_o], axis=1)
    b3 = jnp.concatenate([b_i, b_c, b_o]).reshape(1, 3 * OUT)
    h_new, c_new = _tc_gates(x, agg2, hist, wl3, wr3, b3)
    return (h_new, c_new)
